# Initial kernel scaffold; baseline (speedup 1.0000x reference)
import functools

import jax
import jax.numpy as jnp
from jax import lax
from jax.experimental import pallas as pl
from jax.experimental.pallas import tpu as pltpu
from jax.experimental.pallas import tpu_sc as plsc

_N = 10000
_E = 320000
_D = 128
_NC = 2
_NS = 16
_NW = _NC * _NS
_EPW = _E // _NW
_CH = 128
_NFULL = _EPW // _CH
_TAIL = _EPW - _NFULL * _CH
_ZCH = 624
_ZLAST = _N - 15 * _ZCH
_RPT = _N // _NS

_MESH = plsc.VectorSubcoreMesh(core_axis_name="c", subcore_axis_name="s")


@functools.partial(
    pl.kernel,
    out_type=jax.ShapeDtypeStruct((2, 2, _N), jnp.float32),
    mesh=_MESH,
    scratch_types=[
        pltpu.VMEM((_CH,), jnp.int32),
        pltpu.VMEM((_CH,), jnp.int32),
        pltpu.VMEM((_TAIL,), jnp.int32),
        pltpu.VMEM((_TAIL,), jnp.int32),
        pltpu.VMEM((_ZLAST,), jnp.float32),
        pltpu.VMEM_SHARED((_N,), jnp.float32),
        pltpu.VMEM_SHARED((_N,), jnp.float32),
    ],
)
def _deg_kernel(edges, out, srcb, dstb, srct, dstt, valb, shdego, shdegi):
    c = lax.axis_index("c")
    s = lax.axis_index("s")
    g = c * _NS + s

    @pl.loop(0, _ZLAST // 16)
    def _(i):
        valb[pl.ds(i * 16, 16)] = jnp.zeros((16,), jnp.float32)

    off = s * _ZCH

    @pl.when(s < 15)
    def _():
        pltpu.sync_copy(valb.at[pl.ds(0, _ZCH)], shdego.at[pl.ds(off, _ZCH)])
        pltpu.sync_copy(valb.at[pl.ds(0, _ZCH)], shdegi.at[pl.ds(off, _ZCH)])

    @pl.when(s == 15)
    def _():
        pltpu.sync_copy(valb.at[pl.ds(0, _ZLAST)], shdego.at[pl.ds(off, _ZLAST)])
        pltpu.sync_copy(valb.at[pl.ds(0, _ZLAST)], shdegi.at[pl.ds(off, _ZLAST)])

    plsc.subcore_barrier()

    @pl.loop(0, _ZLAST // 16)
    def _(i):
        valb[pl.ds(i * 16, 16)] = jnp.ones((16,), jnp.float32)

    base0 = g * _EPW

    @pl.loop(0, _NFULL)
    def _(j):
        base = base0 + j * _CH
        pltpu.sync_copy(edges.at[0, pl.ds(base, _CH)], srcb)
        pltpu.sync_copy(edges.at[1, pl.ds(base, _CH)], dstb)
        pltpu.sync_copy(valb.at[pl.ds(0, _CH)], shdego.at[srcb], add=True)
        pltpu.sync_copy(valb.at[pl.ds(0, _CH)], shdegi.at[dstb], add=True)

    tbase = base0 + _NFULL * _CH
    pltpu.sync_copy(edges.at[0, pl.ds(tbase, _TAIL)], srct)
    pltpu.sync_copy(edges.at[1, pl.ds(tbase, _TAIL)], dstt)
    pltpu.sync_copy(valb.at[pl.ds(0, _TAIL)], shdego.at[srct], add=True)
    pltpu.sync_copy(valb.at[pl.ds(0, _TAIL)], shdegi.at[dstt], add=True)

    plsc.subcore_barrier()

    @pl.when(s < 15)
    def _():
        pltpu.sync_copy(shdego.at[pl.ds(off, _ZCH)], out.at[0, c, pl.ds(off, _ZCH)])
        pltpu.sync_copy(shdegi.at[pl.ds(off, _ZCH)], out.at[1, c, pl.ds(off, _ZCH)])

    @pl.when(s == 15)
    def _():
        pltpu.sync_copy(shdego.at[pl.ds(off, _ZLAST)], out.at[0, c, pl.ds(off, _ZLAST)])
        pltpu.sync_copy(shdegi.at[pl.ds(off, _ZLAST)], out.at[1, c, pl.ds(off, _ZLAST)])


@functools.partial(
    pl.kernel,
    out_type=jax.ShapeDtypeStruct((2, _N, _D), jnp.float32),
    mesh=_MESH,
    scratch_types=[
        pltpu.VMEM((_CH,), jnp.int32),
        pltpu.VMEM((_CH,), jnp.int32),
        pltpu.VMEM((_TAIL,), jnp.int32),
        pltpu.VMEM((_TAIL,), jnp.int32),
        pltpu.VMEM((_CH, _D), jnp.float32),
        pltpu.VMEM((_TAIL, _D), jnp.float32),
        pltpu.VMEM_SHARED((_N, _D), jnp.float32),
        pltpu.SemaphoreType.DMA,
    ],
)
def _agg_kernel(t, edges, out, srcb, dstb, srct, dstt, rowsb, rowst, acc, sem):
    c = lax.axis_index("c")
    s = lax.axis_index("s")
    g = c * _NS + s

    @pl.loop(0, _CH * (_D // 16))
    def _(k):
        i = k // (_D // 16)
        j = k % (_D // 16)
        rowsb[i, pl.ds(j * 16, 16)] = jnp.zeros((16,), jnp.float32)

    @pl.loop(0, 5)
    def _(k):
        pltpu.sync_copy(
            rowsb.at[pl.ds(0, _RPT // 5)],
            acc.at[pl.ds(s * _RPT + k * (_RPT // 5), _RPT // 5)],
        )

    plsc.subcore_barrier()

    base0 = g * _EPW

    @pl.loop(0, _NFULL)
    def _(j):
        base = base0 + j * _CH
        pltpu.sync_copy(edges.at[0, pl.ds(base, _CH)], srcb)
        pltpu.sync_copy(edges.at[1, pl.ds(base, _CH)], dstb)
        pltpu.async_copy(t.at[srcb], rowsb, sem).wait()
        pltpu.sync_copy(rowsb, acc.at[dstb], add=True)

    tbase = base0 + _NFULL * _CH
    pltpu.sync_copy(edges.at[0, pl.ds(tbase, _TAIL)], srct)
    pltpu.sync_copy(edges.at[1, pl.ds(tbase, _TAIL)], dstt)
    pltpu.async_copy(t.at[srct], rowst, sem).wait()
    pltpu.sync_copy(rowst, acc.at[dstt], add=True)

    plsc.subcore_barrier()

    pltpu.sync_copy(
        acc.at[pl.ds(s * _RPT, _RPT)], out.at[c, pl.ds(s * _RPT, _RPT)]
    )


_BLK = 1000


def _norm_src(deg_ref):
    d = deg_ref[:, 0:1] + deg_ref[:, 1:2]
    return lax.rsqrt(jnp.maximum(d, 1.0))


def _norm_dst(deg_ref):
    d = deg_ref[:, 2:3] + deg_ref[:, 3:4]
    return lax.rsqrt(jnp.maximum(d, 1.0))


def _mm1_body(x_ref, deg_ref, w_ref, o_ref):
    h = x_ref[...] * _norm_src(deg_ref)
    o_ref[...] = jnp.dot(h, w_ref[...], preferred_element_type=jnp.float32)


def _mid_body(agg_ref, deg_ref, b_ref, w_ref, o_ref):
    p = agg_ref[0] + agg_ref[1]
    h = jnp.maximum(p * _norm_dst(deg_ref) + b_ref[...], 0.0)
    h = h * _norm_src(deg_ref)
    o_ref[...] = jnp.dot(h, w_ref[...], preferred_element_type=jnp.float32)


def _final_body(agg_ref, deg_ref, b_ref, o_ref):
    p = agg_ref[0] + agg_ref[1]
    o_ref[...] = jnp.maximum(p * _norm_dst(deg_ref) + b_ref[...], 0.0)


_deg_spec = pl.BlockSpec((_BLK, 4), lambda i: (i, 0))
_row_spec = pl.BlockSpec((_BLK, _D), lambda i: (i, 0))
_agg_spec = pl.BlockSpec((2, _BLK, _D), lambda i: (0, i, 0))
_w_spec = pl.BlockSpec((_D, _D), lambda i: (0, 0))
_b_spec = pl.BlockSpec((1, _D), lambda i: (0, 0))
_out_shape = jax.ShapeDtypeStruct((_N, _D), jnp.float32)

_mm1_call = pl.pallas_call(
    _mm1_body,
    grid=(_N // _BLK,),
    in_specs=[_row_spec, _deg_spec, _w_spec],
    out_specs=_row_spec,
    out_shape=_out_shape,
)

_mid_call = pl.pallas_call(
    _mid_body,
    grid=(_N // _BLK,),
    in_specs=[_agg_spec, _deg_spec, _b_spec, _w_spec],
    out_specs=_row_spec,
    out_shape=_out_shape,
)

_final_call = pl.pallas_call(
    _final_body,
    grid=(_N // _BLK,),
    in_specs=[_agg_spec, _deg_spec, _b_spec],
    out_specs=_row_spec,
    out_shape=_out_shape,
)


@jax.jit
def kernel(x, edge_index, W1, b1, W2, b2):
    degs = _deg_kernel(edge_index)
    degs = jnp.transpose(degs.reshape(4, _N))
    t1 = _mm1_call(x, degs, W1)
    agg1 = _agg_kernel(t1, edge_index)
    t2 = _mid_call(agg1, degs, b1.reshape(1, _D), W2)
    agg2 = _agg_kernel(t2, edge_index)
    return _final_call(agg2, degs, b2.reshape(1, _D))


# speedup vs baseline: 6.1115x; 6.1115x over previous
import functools

import jax
import jax.numpy as jnp
from jax import lax
from jax.experimental import pallas as pl
from jax.experimental.pallas import tpu as pltpu
from jax.experimental.pallas import tpu_sc as plsc

_N = 10000
_E = 320000
_D = 128
_NC = 2
_NS = 16
_NW = _NC * _NS
_EPW = _E // _NW
_CH = 128
_NFULL = _EPW // _CH
_TAIL = _EPW - _NFULL * _CH
_ZCH = 624
_ZLAST = _N - 15 * _ZCH
_RPT = _N // _NS

_MESH = plsc.VectorSubcoreMesh(core_axis_name="c", subcore_axis_name="s")


@functools.partial(
    pl.kernel,
    out_type=[jax.ShapeDtypeStruct((_N,), jnp.float32)] * 4,
    mesh=_MESH,
    scratch_types=[
        pltpu.VMEM((_CH,), jnp.int32),
        pltpu.VMEM((_CH,), jnp.int32),
        pltpu.VMEM((_TAIL,), jnp.int32),
        pltpu.VMEM((_TAIL,), jnp.int32),
        pltpu.VMEM((_ZLAST,), jnp.float32),
        pltpu.VMEM_SHARED((_N,), jnp.float32),
        pltpu.VMEM_SHARED((_N,), jnp.float32),
    ],
)
def _deg_kernel(esrc, edst, out_o0, out_o1, out_i0, out_i1,
                srcb, dstb, srct, dstt, valb, shdego, shdegi):
    c = lax.axis_index("c")
    s = lax.axis_index("s")
    g = c * _NS + s

    @pl.loop(0, _ZLAST // 16)
    def _(i):
        valb[pl.ds(i * 16, 16)] = jnp.zeros((16,), jnp.float32)

    off = s * _ZCH

    @pl.when(s < 15)
    def _():
        pltpu.sync_copy(valb.at[pl.ds(0, _ZCH)], shdego.at[pl.ds(off, _ZCH)])
        pltpu.sync_copy(valb.at[pl.ds(0, _ZCH)], shdegi.at[pl.ds(off, _ZCH)])

    @pl.when(s == 15)
    def _():
        pltpu.sync_copy(valb.at[pl.ds(0, _ZLAST)], shdego.at[pl.ds(off, _ZLAST)])
        pltpu.sync_copy(valb.at[pl.ds(0, _ZLAST)], shdegi.at[pl.ds(off, _ZLAST)])

    plsc.subcore_barrier()

    @pl.loop(0, _ZLAST // 16)
    def _(i):
        valb[pl.ds(i * 16, 16)] = jnp.ones((16,), jnp.float32)

    base0 = g * _EPW

    @pl.loop(0, _NFULL)
    def _(j):
        base = base0 + j * _CH
        pltpu.sync_copy(esrc.at[pl.ds(base, _CH)], srcb)
        pltpu.sync_copy(edst.at[pl.ds(base, _CH)], dstb)
        pltpu.sync_copy(valb.at[pl.ds(0, _CH)], shdego.at[srcb], add=True)
        pltpu.sync_copy(valb.at[pl.ds(0, _CH)], shdegi.at[dstb], add=True)

    tbase = base0 + _NFULL * _CH
    pltpu.sync_copy(esrc.at[pl.ds(tbase, _TAIL)], srct)
    pltpu.sync_copy(edst.at[pl.ds(tbase, _TAIL)], dstt)
    pltpu.sync_copy(valb.at[pl.ds(0, _TAIL)], shdego.at[srct], add=True)
    pltpu.sync_copy(valb.at[pl.ds(0, _TAIL)], shdegi.at[dstt], add=True)

    plsc.subcore_barrier()

    sz_small = s < 15

    def _emit(oref, iref):
        @pl.when(sz_small)
        def _():
            pltpu.sync_copy(shdego.at[pl.ds(off, _ZCH)], valb.at[pl.ds(0, _ZCH)])
            pltpu.sync_copy(valb.at[pl.ds(0, _ZCH)], oref.at[pl.ds(off, _ZCH)])
            pltpu.sync_copy(shdegi.at[pl.ds(off, _ZCH)], valb.at[pl.ds(0, _ZCH)])
            pltpu.sync_copy(valb.at[pl.ds(0, _ZCH)], iref.at[pl.ds(off, _ZCH)])

        @pl.when(jnp.logical_not(sz_small))
        def _():
            pltpu.sync_copy(shdego.at[pl.ds(off, _ZLAST)], valb.at[pl.ds(0, _ZLAST)])
            pltpu.sync_copy(valb.at[pl.ds(0, _ZLAST)], oref.at[pl.ds(off, _ZLAST)])
            pltpu.sync_copy(shdegi.at[pl.ds(off, _ZLAST)], valb.at[pl.ds(0, _ZLAST)])
            pltpu.sync_copy(valb.at[pl.ds(0, _ZLAST)], iref.at[pl.ds(off, _ZLAST)])

    @pl.when(c == 0)
    def _():
        _emit(out_o0, out_i0)

    @pl.when(c == 1)
    def _():
        _emit(out_o1, out_i1)


@functools.partial(
    pl.kernel,
    out_type=jax.ShapeDtypeStruct((2, _N, _D), jnp.float32),
    mesh=_MESH,
    scratch_types=[
        pltpu.VMEM((_CH,), jnp.int32),
        pltpu.VMEM((_CH,), jnp.int32),
        pltpu.VMEM((_TAIL,), jnp.int32),
        pltpu.VMEM((_TAIL,), jnp.int32),
        pltpu.VMEM((_CH, _D), jnp.float32),
        pltpu.VMEM((_TAIL, _D), jnp.float32),
        pltpu.VMEM_SHARED((_N, _D), jnp.float32),
        pltpu.SemaphoreType.DMA,
    ],
)
def _agg_kernel(t, esrc, edst, out, srcb, dstb, srct, dstt, rowsb, rowst, acc, sem):
    c = lax.axis_index("c")
    s = lax.axis_index("s")
    g = c * _NS + s

    @pl.loop(0, _CH * (_D // 16))
    def _(k):
        i = k // (_D // 16)
        j = k % (_D // 16)
        rowsb[i, pl.ds(j * 16, 16)] = jnp.zeros((16,), jnp.float32)

    @pl.when(s < 15)
    def _():
        @pl.loop(0, 6)
        def _(k):
            pltpu.sync_copy(
                rowsb.at[pl.ds(0, 104)],
                acc.at[pl.ds(s * _ZCH + k * 104, 104)],
            )

    @pl.when(s == 15)
    def _():
        @pl.loop(0, 5)
        def _(k):
            pltpu.sync_copy(
                rowsb.at[pl.ds(0, _CH)],
                acc.at[pl.ds(15 * _ZCH + k * _CH, _CH)],
            )

    plsc.subcore_barrier()

    base0 = g * _EPW

    @pl.loop(0, _NFULL)
    def _(j):
        base = base0 + j * _CH
        pltpu.sync_copy(esrc.at[pl.ds(base, _CH)], srcb)
        pltpu.sync_copy(edst.at[pl.ds(base, _CH)], dstb)
        pltpu.async_copy(t.at[srcb], rowsb, sem).wait()
        pltpu.sync_copy(rowsb, acc.at[dstb], add=True)

    tbase = base0 + _NFULL * _CH
    pltpu.sync_copy(esrc.at[pl.ds(tbase, _TAIL)], srct)
    pltpu.sync_copy(edst.at[pl.ds(tbase, _TAIL)], dstt)
    pltpu.async_copy(t.at[srct], rowst, sem).wait()
    pltpu.sync_copy(rowst, acc.at[dstt], add=True)

    plsc.subcore_barrier()

    @pl.when(s < 15)
    def _():
        @pl.loop(0, 6)
        def _(k):
            roff = s * _ZCH + k * 104
            pltpu.sync_copy(acc.at[pl.ds(roff, 104)], rowsb.at[pl.ds(0, 104)])
            pltpu.sync_copy(rowsb.at[pl.ds(0, 104)], out.at[c, pl.ds(roff, 104)])

    @pl.when(s == 15)
    def _():
        @pl.loop(0, 5)
        def _(k):
            roff = 15 * _ZCH + k * _CH
            pltpu.sync_copy(acc.at[pl.ds(roff, _CH)], rowsb.at[pl.ds(0, _CH)])
            pltpu.sync_copy(rowsb.at[pl.ds(0, _CH)], out.at[c, pl.ds(roff, _CH)])


_BLK = 1000


def _norm_src(deg_ref):
    d = deg_ref[:, 0:1] + deg_ref[:, 1:2]
    return lax.rsqrt(jnp.maximum(d, 1.0))


def _norm_dst(deg_ref):
    d = deg_ref[:, 2:3] + deg_ref[:, 3:4]
    return lax.rsqrt(jnp.maximum(d, 1.0))


def _mm1_body(x_ref, deg_ref, w_ref, o_ref):
    h = x_ref[...] * _norm_src(deg_ref)
    o_ref[...] = jnp.dot(h, w_ref[...], preferred_element_type=jnp.float32)


def _mid_body(agg_ref, deg_ref, b_ref, w_ref, o_ref):
    p = agg_ref[0] + agg_ref[1]
    h = jnp.maximum(p * _norm_dst(deg_ref) + b_ref[...], 0.0)
    h = h * _norm_src(deg_ref)
    o_ref[...] = jnp.dot(h, w_ref[...], preferred_element_type=jnp.float32)


def _final_body(agg_ref, deg_ref, b_ref, o_ref):
    p = agg_ref[0] + agg_ref[1]
    o_ref[...] = jnp.maximum(p * _norm_dst(deg_ref) + b_ref[...], 0.0)


_deg_spec = pl.BlockSpec((_BLK, 4), lambda i: (i, 0))
_row_spec = pl.BlockSpec((_BLK, _D), lambda i: (i, 0))
_agg_spec = pl.BlockSpec((2, _BLK, _D), lambda i: (0, i, 0))
_w_spec = pl.BlockSpec((_D, _D), lambda i: (0, 0))
_b_spec = pl.BlockSpec((1, _D), lambda i: (0, 0))
_out_shape = jax.ShapeDtypeStruct((_N, _D), jnp.float32)

_mm1_call = pl.pallas_call(
    _mm1_body,
    grid=(_N // _BLK,),
    in_specs=[_row_spec, _deg_spec, _w_spec],
    out_specs=_row_spec,
    out_shape=_out_shape,
)

_mid_call = pl.pallas_call(
    _mid_body,
    grid=(_N // _BLK,),
    in_specs=[_agg_spec, _deg_spec, _b_spec, _w_spec],
    out_specs=_row_spec,
    out_shape=_out_shape,
)

_final_call = pl.pallas_call(
    _final_body,
    grid=(_N // _BLK,),
    in_specs=[_agg_spec, _deg_spec, _b_spec],
    out_specs=_row_spec,
    out_shape=_out_shape,
)


@jax.jit
def kernel(x, edge_index, W1, b1, W2, b2):
    esrc = edge_index[0]
    edst = edge_index[1]
    do0, do1, di0, di1 = _deg_kernel(esrc, edst)
    degs = jnp.stack([do0, do1, di0, di1], axis=1)
    t1 = _mm1_call(x, degs, W1)
    agg1 = _agg_kernel(t1, esrc, edst)
    t2 = _mid_call(agg1, degs, b1.reshape(1, _D), W2)
    agg2 = _agg_kernel(t2, esrc, edst)
    return _final_call(agg2, degs, b2.reshape(1, _D))


# speedup vs baseline: 9.4180x; 1.5410x over previous
import functools

import jax
import jax.numpy as jnp
from jax import lax
from jax.experimental import pallas as pl
from jax.experimental.pallas import tpu as pltpu
from jax.experimental.pallas import tpu_sc as plsc

_N = 10000
_E = 320000
_D = 128
_H = _D // 2
_NC = 2
_NS = 16
_NW = _NC * _NS
_CH = 80

_DEPW = _E // _NW
_DNCH = _DEPW // _CH

_NST = 5
_SCH = _DNCH // _NST

_ZCH = 624
_ZLAST = _N - 15 * _ZCH

_MESH = plsc.VectorSubcoreMesh(core_axis_name="c", subcore_axis_name="s")


@functools.partial(
    pl.kernel,
    out_type=[jax.ShapeDtypeStruct((_N,), jnp.float32)] * 4,
    mesh=_MESH,
    scratch_types=[
        pltpu.VMEM((_DNCH, _CH), jnp.int32),
        pltpu.VMEM((_DNCH, _CH), jnp.int32),
        pltpu.VMEM((_ZLAST,), jnp.float32),
        pltpu.VMEM_SHARED((_N,), jnp.float32),
        pltpu.VMEM_SHARED((_N,), jnp.float32),
        pltpu.SemaphoreType.DMA,
        pltpu.SemaphoreType.DMA,
    ],
)
def _deg_kernel(esrc3, edst3, out_o0, out_o1, out_i0, out_i1,
                srcb, dstb, valb, shdego, shdegi, sem0, sem1):
    c = lax.axis_index("c")
    s = lax.axis_index("s")
    g = c * _NS + s

    @pl.loop(0, _ZLAST // 16)
    def _(i):
        valb[pl.ds(i * 16, 16)] = jnp.zeros((16,), jnp.float32)

    off = s * _ZCH

    @pl.when(s < 15)
    def _():
        pltpu.sync_copy(valb.at[pl.ds(0, _ZCH)], shdego.at[pl.ds(off, _ZCH)])
        pltpu.sync_copy(valb.at[pl.ds(0, _ZCH)], shdegi.at[pl.ds(off, _ZCH)])

    @pl.when(s == 15)
    def _():
        pltpu.sync_copy(valb.at[pl.ds(0, _ZLAST)], shdego.at[pl.ds(off, _ZLAST)])
        pltpu.sync_copy(valb.at[pl.ds(0, _ZLAST)], shdegi.at[pl.ds(off, _ZLAST)])

    pltpu.sync_copy(esrc3.at[g], srcb)
    pltpu.sync_copy(edst3.at[g], dstb)

    plsc.subcore_barrier()

    @pl.loop(0, _ZLAST // 16)
    def _(i):
        valb[pl.ds(i * 16, 16)] = jnp.ones((16,), jnp.float32)

    ones = valb.at[pl.ds(0, _CH)]
    pltpu.async_copy(ones, shdego.at[srcb.at[0]], sem0, add=True)
    pltpu.async_copy(ones, shdegi.at[dstb.at[0]], sem1, add=True)

    @pl.loop(0, _DNCH)
    def _(j):
        @pl.when(j + 1 < _DNCH)
        def _():
            pltpu.async_copy(ones, shdego.at[srcb.at[j + 1]], sem0, add=True)
            pltpu.async_copy(ones, shdegi.at[dstb.at[j + 1]], sem1, add=True)

        pltpu.make_async_copy(ones, shdego.at[srcb.at[j]], sem0).wait()
        pltpu.make_async_copy(ones, shdegi.at[dstb.at[j]], sem1).wait()

    plsc.subcore_barrier()

    sz_small = s < 15

    def _emit(oref, iref):
        @pl.when(sz_small)
        def _():
            pltpu.sync_copy(shdego.at[pl.ds(off, _ZCH)], valb.at[pl.ds(0, _ZCH)])
            pltpu.sync_copy(valb.at[pl.ds(0, _ZCH)], oref.at[pl.ds(off, _ZCH)])
            pltpu.sync_copy(shdegi.at[pl.ds(off, _ZCH)], valb.at[pl.ds(0, _ZCH)])
            pltpu.sync_copy(valb.at[pl.ds(0, _ZCH)], iref.at[pl.ds(off, _ZCH)])

        @pl.when(jnp.logical_not(sz_small))
        def _():
            pltpu.sync_copy(shdego.at[pl.ds(off, _ZLAST)], valb.at[pl.ds(0, _ZLAST)])
            pltpu.sync_copy(valb.at[pl.ds(0, _ZLAST)], oref.at[pl.ds(off, _ZLAST)])
            pltpu.sync_copy(shdegi.at[pl.ds(off, _ZLAST)], valb.at[pl.ds(0, _ZLAST)])
            pltpu.sync_copy(valb.at[pl.ds(0, _ZLAST)], iref.at[pl.ds(off, _ZLAST)])

    @pl.when(c == 0)
    def _():
        _emit(out_o0, out_i0)

    @pl.when(c == 1)
    def _():
        _emit(out_o1, out_i1)


@functools.partial(
    pl.kernel,
    out_type=jax.ShapeDtypeStruct((2, _N, _D), jnp.float32),
    mesh=_MESH,
    scratch_types=[
        pltpu.VMEM((_SCH, _CH), jnp.int32),
        pltpu.VMEM((_SCH, _CH), jnp.int32),
        pltpu.VMEM((_CH, _D), jnp.float32),
        pltpu.VMEM((_CH, _D), jnp.float32),
        pltpu.VMEM_SHARED((_N, _D), jnp.float32),
        pltpu.SemaphoreType.DMA,
        pltpu.SemaphoreType.DMA,
    ],
)
def _agg_kernel(t, esrc4, edst4, out, srcb, dstb, rows0, rows1, acc, sem0, sem1):
    c = lax.axis_index("c")
    s = lax.axis_index("s")
    g = c * _NS + s

    @pl.loop(0, _CH * (_D // 16))
    def _(k):
        i = k // (_D // 16)
        j = k % (_D // 16)
        rows0[i, pl.ds(j * 16, 16)] = jnp.zeros((16,), jnp.float32)

    @pl.when(s < 15)
    def _():
        @pl.loop(0, 7)
        def _(k):
            pltpu.sync_copy(
                rows0.at[pl.ds(0, _CH)],
                acc.at[pl.ds(s * _ZCH + k * _CH, _CH)],
            )

        pltpu.sync_copy(
            rows0.at[pl.ds(0, 64)],
            acc.at[pl.ds(s * _ZCH + 560, 64)],
        )

    @pl.when(s == 15)
    def _():
        @pl.loop(0, 8)
        def _(k):
            pltpu.sync_copy(
                rows0.at[pl.ds(0, _CH)],
                acc.at[pl.ds(15 * _ZCH + k * _CH, _CH)],
            )

    plsc.subcore_barrier()

    @pl.loop(0, _NST)
    def _(st):
        pltpu.sync_copy(esrc4.at[g, st], srcb)
        pltpu.sync_copy(edst4.at[g, st], dstb)

        pltpu.async_copy(t.at[srcb.at[0]], rows0, sem0)

        @pl.loop(0, _SCH)
        def _(j):
            even = j % 2 == 0

            @pl.when(even)
            def _():
                pltpu.make_async_copy(t.at[srcb.at[j]], rows0, sem0).wait()

                @pl.when(j + 1 < _SCH)
                def _():
                    pltpu.async_copy(t.at[srcb.at[j + 1]], rows1, sem1)

                pltpu.sync_copy(rows0, acc.at[dstb.at[j]], add=True)

            @pl.when(jnp.logical_not(even))
            def _():
                pltpu.make_async_copy(t.at[srcb.at[j]], rows1, sem1).wait()

                @pl.when(j + 1 < _SCH)
                def _():
                    pltpu.async_copy(t.at[srcb.at[j + 1]], rows0, sem0)

                pltpu.sync_copy(rows1, acc.at[dstb.at[j]], add=True)

    plsc.subcore_barrier()

    @pl.when(s < 15)
    def _():
        @pl.loop(0, 7)
        def _(k):
            roff = s * _ZCH + k * _CH
            pltpu.sync_copy(acc.at[pl.ds(roff, _CH)], rows0.at[pl.ds(0, _CH)])
            pltpu.sync_copy(rows0.at[pl.ds(0, _CH)], out.at[c, pl.ds(roff, _CH)])

        tl = s * _ZCH + 560
        pltpu.sync_copy(acc.at[pl.ds(tl, 64)], rows0.at[pl.ds(0, 64)])
        pltpu.sync_copy(rows0.at[pl.ds(0, 64)], out.at[c, pl.ds(tl, 64)])

    @pl.when(s == 15)
    def _():
        @pl.loop(0, 8)
        def _(k):
            roff = 15 * _ZCH + k * _CH
            pltpu.sync_copy(acc.at[pl.ds(roff, _CH)], rows0.at[pl.ds(0, _CH)])
            pltpu.sync_copy(rows0.at[pl.ds(0, _CH)], out.at[c, pl.ds(roff, _CH)])


_BLK = 1000


def _norm_src(deg_ref):
    d = deg_ref[:, 0:1] + deg_ref[:, 1:2]
    return lax.rsqrt(jnp.maximum(d, 1.0))


def _norm_dst(deg_ref):
    d = deg_ref[:, 2:3] + deg_ref[:, 3:4]
    return lax.rsqrt(jnp.maximum(d, 1.0))


def _mm1_body(x_ref, deg_ref, w_ref, o_ref):
    h = x_ref[...] * _norm_src(deg_ref)
    o_ref[...] = jnp.dot(h, w_ref[...], preferred_element_type=jnp.float32)


def _mid_body(agg_ref, deg_ref, b_ref, w_ref, o_ref):
    p = agg_ref[0] + agg_ref[1]
    h = jnp.maximum(p * _norm_dst(deg_ref) + b_ref[...], 0.0)
    h = h * _norm_src(deg_ref)
    o_ref[...] = jnp.dot(h, w_ref[...], preferred_element_type=jnp.float32)


def _final_body(agg_ref, deg_ref, b_ref, o_ref):
    p = agg_ref[0] + agg_ref[1]
    o_ref[...] = jnp.maximum(p * _norm_dst(deg_ref) + b_ref[...], 0.0)


_deg_spec = pl.BlockSpec((_BLK, 4), lambda i: (i, 0))
_row_spec = pl.BlockSpec((_BLK, _D), lambda i: (i, 0))
_agg_spec = pl.BlockSpec((2, _BLK, _D), lambda i: (0, i, 0))
_w_spec = pl.BlockSpec((_D, _D), lambda i: (0, 0))
_b_spec = pl.BlockSpec((1, _D), lambda i: (0, 0))
_out_full = jax.ShapeDtypeStruct((_N, _D), jnp.float32)

_mm1_call = pl.pallas_call(
    _mm1_body,
    grid=(_N // _BLK,),
    in_specs=[_row_spec, _deg_spec, _w_spec],
    out_specs=_row_spec,
    out_shape=_out_full,
)

_mid_call = pl.pallas_call(
    _mid_body,
    grid=(_N // _BLK,),
    in_specs=[_agg_spec, _deg_spec, _b_spec, _w_spec],
    out_specs=_row_spec,
    out_shape=_out_full,
)

_final_call = pl.pallas_call(
    _final_body,
    grid=(_N // _BLK,),
    in_specs=[_agg_spec, _deg_spec, _b_spec],
    out_specs=_row_spec,
    out_shape=_out_full,
)


@jax.jit
def kernel(x, edge_index, W1, b1, W2, b2):
    e32 = edge_index.reshape(2, _NW, _DNCH, _CH)
    e4 = edge_index.reshape(2, _NW, _NST, _SCH, _CH)
    do0, do1, di0, di1 = _deg_kernel(e32[0], e32[1])
    degs = jnp.stack([do0, do1, di0, di1], axis=1)
    t1 = _mm1_call(x, degs, W1)
    agg1 = _agg_kernel(t1, e4[0], e4[1])
    t2 = _mid_call(agg1, degs, b1.reshape(1, _D), W2)
    agg2 = _agg_kernel(t2, e4[0], e4[1])
    return _final_call(agg2, degs, b2.reshape(1, _D))


# speedup vs baseline: 12.5060x; 1.3279x over previous
import functools

import jax
import jax.numpy as jnp
from jax import lax
from jax.experimental import pallas as pl
from jax.experimental.pallas import tpu as pltpu
from jax.experimental.pallas import tpu_sc as plsc

_N = 10000
_E = 320000
_D = 128
_H = _D // 2
_NC = 2
_NS = 16
_NW = _NC * _NS
_CH = 80

_DEPW = _E // _NW
_DNCH = _DEPW // _CH

_NST = 5
_SCH = _DNCH // _NST

_ZCH = 624
_ZLAST = _N - 15 * _ZCH

_MESH = plsc.VectorSubcoreMesh(core_axis_name="c", subcore_axis_name="s")


@functools.partial(
    pl.kernel,
    out_type=[jax.ShapeDtypeStruct((_N,), jnp.float32)] * 4,
    mesh=_MESH,
    scratch_types=[
        pltpu.VMEM((_DNCH, _CH), jnp.int32),
        pltpu.VMEM((_DNCH, _CH), jnp.int32),
        pltpu.VMEM((_ZLAST,), jnp.float32),
        pltpu.VMEM_SHARED((_N,), jnp.float32),
        pltpu.VMEM_SHARED((_N,), jnp.float32),
        pltpu.SemaphoreType.DMA,
        pltpu.SemaphoreType.DMA,
    ],
)
def _deg_kernel(esrc3, edst3, out_o0, out_o1, out_i0, out_i1,
                srcb, dstb, valb, shdego, shdegi, sem0, sem1):
    c = lax.axis_index("c")
    s = lax.axis_index("s")
    g = c * _NS + s

    @pl.loop(0, _ZLAST // 16)
    def _(i):
        valb[pl.ds(i * 16, 16)] = jnp.zeros((16,), jnp.float32)

    off = s * _ZCH

    @pl.when(s < 15)
    def _():
        pltpu.sync_copy(valb.at[pl.ds(0, _ZCH)], shdego.at[pl.ds(off, _ZCH)])
        pltpu.sync_copy(valb.at[pl.ds(0, _ZCH)], shdegi.at[pl.ds(off, _ZCH)])

    @pl.when(s == 15)
    def _():
        pltpu.sync_copy(valb.at[pl.ds(0, _ZLAST)], shdego.at[pl.ds(off, _ZLAST)])
        pltpu.sync_copy(valb.at[pl.ds(0, _ZLAST)], shdegi.at[pl.ds(off, _ZLAST)])

    pltpu.sync_copy(esrc3.at[g], srcb)
    pltpu.sync_copy(edst3.at[g], dstb)

    plsc.subcore_barrier()

    @pl.loop(0, _ZLAST // 16)
    def _(i):
        valb[pl.ds(i * 16, 16)] = jnp.ones((16,), jnp.float32)

    ones = valb.at[pl.ds(0, _CH)]
    pltpu.async_copy(ones, shdego.at[srcb.at[0]], sem0, add=True)
    pltpu.async_copy(ones, shdegi.at[dstb.at[0]], sem1, add=True)

    @pl.loop(0, _DNCH)
    def _(j):
        @pl.when(j + 1 < _DNCH)
        def _():
            pltpu.async_copy(ones, shdego.at[srcb.at[j + 1]], sem0, add=True)
            pltpu.async_copy(ones, shdegi.at[dstb.at[j + 1]], sem1, add=True)

        pltpu.make_async_copy(ones, shdego.at[srcb.at[j]], sem0).wait()
        pltpu.make_async_copy(ones, shdegi.at[dstb.at[j]], sem1).wait()

    plsc.subcore_barrier()

    sz_small = s < 15

    def _emit(oref, iref):
        @pl.when(sz_small)
        def _():
            pltpu.sync_copy(shdego.at[pl.ds(off, _ZCH)], valb.at[pl.ds(0, _ZCH)])
            pltpu.sync_copy(valb.at[pl.ds(0, _ZCH)], oref.at[pl.ds(off, _ZCH)])
            pltpu.sync_copy(shdegi.at[pl.ds(off, _ZCH)], valb.at[pl.ds(0, _ZCH)])
            pltpu.sync_copy(valb.at[pl.ds(0, _ZCH)], iref.at[pl.ds(off, _ZCH)])

        @pl.when(jnp.logical_not(sz_small))
        def _():
            pltpu.sync_copy(shdego.at[pl.ds(off, _ZLAST)], valb.at[pl.ds(0, _ZLAST)])
            pltpu.sync_copy(valb.at[pl.ds(0, _ZLAST)], oref.at[pl.ds(off, _ZLAST)])
            pltpu.sync_copy(shdegi.at[pl.ds(off, _ZLAST)], valb.at[pl.ds(0, _ZLAST)])
            pltpu.sync_copy(valb.at[pl.ds(0, _ZLAST)], iref.at[pl.ds(off, _ZLAST)])

    @pl.when(c == 0)
    def _():
        _emit(out_o0, out_i0)

    @pl.when(c == 1)
    def _():
        _emit(out_o1, out_i1)


@functools.partial(
    pl.kernel,
    out_type=jax.ShapeDtypeStruct((2, _N, _D), jnp.float32),
    mesh=_MESH,
    scratch_types=[
        pltpu.VMEM((_SCH, _CH), jnp.int32),
        pltpu.VMEM((_SCH, _CH), jnp.int32),
        pltpu.VMEM((_CH, _D), jnp.float32),
        pltpu.VMEM((_CH, _D), jnp.float32),
        pltpu.VMEM((_CH, _D), jnp.float32),
        pltpu.VMEM_SHARED((_N, _D), jnp.float32),
        pltpu.SemaphoreType.DMA,
        pltpu.SemaphoreType.DMA,
        pltpu.SemaphoreType.DMA,
        pltpu.SemaphoreType.DMA,
        pltpu.SemaphoreType.DMA,
        pltpu.SemaphoreType.DMA,
    ],
)
def _agg_kernel(t, esrc4, edst4, out, srcb, dstb, rows0, rows1, rows2, acc,
                gs0, gs1, gs2, ss0, ss1, ss2):
    c = lax.axis_index("c")
    s = lax.axis_index("s")
    g = c * _NS + s

    @pl.loop(0, _CH * (_D // 16))
    def _(k):
        i = k // (_D // 16)
        j = k % (_D // 16)
        rows0[i, pl.ds(j * 16, 16)] = jnp.zeros((16,), jnp.float32)

    @pl.when(s < 15)
    def _():
        @pl.loop(0, 7)
        def _(k):
            pltpu.sync_copy(
                rows0.at[pl.ds(0, _CH)],
                acc.at[pl.ds(s * _ZCH + k * _CH, _CH)],
            )

        pltpu.sync_copy(
            rows0.at[pl.ds(0, 64)],
            acc.at[pl.ds(s * _ZCH + 560, 64)],
        )

    @pl.when(s == 15)
    def _():
        @pl.loop(0, 8)
        def _(k):
            pltpu.sync_copy(
                rows0.at[pl.ds(0, _CH)],
                acc.at[pl.ds(15 * _ZCH + k * _CH, _CH)],
            )

    plsc.subcore_barrier()

    rows = (rows0, rows1, rows2)
    gsem = (gs0, gs1, gs2)
    ssem = (ss0, ss1, ss2)

    @pl.loop(0, _NST)
    def _(st):
        pltpu.sync_copy(esrc4.at[g, st], srcb)
        pltpu.sync_copy(edst4.at[g, st], dstb)

        pltpu.async_copy(t.at[srcb.at[0]], rows[0], gsem[0])
        pltpu.async_copy(t.at[srcb.at[1]], rows[1], gsem[1])

        @pl.loop(0, _SCH)
        def _(j):
            for b in range(3):
                @pl.when(j % 3 == b)
                def _(b=b):
                    pltpu.make_async_copy(t.at[srcb.at[j]], rows[b], gsem[b]).wait()
                    pltpu.async_copy(rows[b], acc.at[dstb.at[j]], ssem[b], add=True)

                    @pl.when(j + 2 < _SCH)
                    def _(b=b):
                        nb = (b + 2) % 3

                        @pl.when(j >= 1)
                        def _(nb=nb):
                            pltpu.make_async_copy(
                                rows[nb], acc.at[dstb.at[j - 1]], ssem[nb]
                            ).wait()

                        pltpu.async_copy(t.at[srcb.at[j + 2]], rows[nb], gsem[nb])

        for k in (_SCH - 3, _SCH - 2, _SCH - 1):
            pltpu.make_async_copy(rows[k % 3], acc.at[dstb.at[k]], ssem[k % 3]).wait()

    plsc.subcore_barrier()

    @pl.when(s < 15)
    def _():
        @pl.loop(0, 7)
        def _(k):
            roff = s * _ZCH + k * _CH
            pltpu.sync_copy(acc.at[pl.ds(roff, _CH)], rows0.at[pl.ds(0, _CH)])
            pltpu.sync_copy(rows0.at[pl.ds(0, _CH)], out.at[c, pl.ds(roff, _CH)])

        tl = s * _ZCH + 560
        pltpu.sync_copy(acc.at[pl.ds(tl, 64)], rows0.at[pl.ds(0, 64)])
        pltpu.sync_copy(rows0.at[pl.ds(0, 64)], out.at[c, pl.ds(tl, 64)])

    @pl.when(s == 15)
    def _():
        @pl.loop(0, 8)
        def _(k):
            roff = 15 * _ZCH + k * _CH
            pltpu.sync_copy(acc.at[pl.ds(roff, _CH)], rows0.at[pl.ds(0, _CH)])
            pltpu.sync_copy(rows0.at[pl.ds(0, _CH)], out.at[c, pl.ds(roff, _CH)])


_BLK = 1000


def _norm_src(deg_ref):
    d = deg_ref[:, 0:1] + deg_ref[:, 1:2]
    return lax.rsqrt(jnp.maximum(d, 1.0))


def _norm_dst(deg_ref):
    d = deg_ref[:, 2:3] + deg_ref[:, 3:4]
    return lax.rsqrt(jnp.maximum(d, 1.0))


def _mm1_body(x_ref, deg_ref, w_ref, o_ref):
    h = x_ref[...] * _norm_src(deg_ref)
    o_ref[...] = jnp.dot(h, w_ref[...], preferred_element_type=jnp.float32)


def _mid_body(agg_ref, deg_ref, b_ref, w_ref, o_ref):
    p = agg_ref[0] + agg_ref[1]
    h = jnp.maximum(p * _norm_dst(deg_ref) + b_ref[...], 0.0)
    h = h * _norm_src(deg_ref)
    o_ref[...] = jnp.dot(h, w_ref[...], preferred_element_type=jnp.float32)


def _final_body(agg_ref, deg_ref, b_ref, o_ref):
    p = agg_ref[0] + agg_ref[1]
    o_ref[...] = jnp.maximum(p * _norm_dst(deg_ref) + b_ref[...], 0.0)


_deg_spec = pl.BlockSpec((_BLK, 4), lambda i: (i, 0))
_row_spec = pl.BlockSpec((_BLK, _D), lambda i: (i, 0))
_agg_spec = pl.BlockSpec((2, _BLK, _D), lambda i: (0, i, 0))
_w_spec = pl.BlockSpec((_D, _D), lambda i: (0, 0))
_b_spec = pl.BlockSpec((1, _D), lambda i: (0, 0))
_out_full = jax.ShapeDtypeStruct((_N, _D), jnp.float32)

_mm1_call = pl.pallas_call(
    _mm1_body,
    grid=(_N // _BLK,),
    in_specs=[_row_spec, _deg_spec, _w_spec],
    out_specs=_row_spec,
    out_shape=_out_full,
)

_mid_call = pl.pallas_call(
    _mid_body,
    grid=(_N // _BLK,),
    in_specs=[_agg_spec, _deg_spec, _b_spec, _w_spec],
    out_specs=_row_spec,
    out_shape=_out_full,
)

_final_call = pl.pallas_call(
    _final_body,
    grid=(_N // _BLK,),
    in_specs=[_agg_spec, _deg_spec, _b_spec],
    out_specs=_row_spec,
    out_shape=_out_full,
)


@jax.jit
def kernel(x, edge_index, W1, b1, W2, b2):
    e32 = edge_index.reshape(2, _NW, _DNCH, _CH)
    e4 = edge_index.reshape(2, _NW, _NST, _SCH, _CH)
    do0, do1, di0, di1 = _deg_kernel(e32[0], e32[1])
    degs = jnp.stack([do0, do1, di0, di1], axis=1)
    t1 = _mm1_call(x, degs, W1)
    agg1 = _agg_kernel(t1, e4[0], e4[1])
    t2 = _mid_call(agg1, degs, b1.reshape(1, _D), W2)
    agg2 = _agg_kernel(t2, e4[0], e4[1])
    return _final_call(agg2, degs, b2.reshape(1, _D))
